# baseline (speedup 1.0000x reference)
import functools

import jax
import jax.numpy as jnp
from jax import lax
from jax.experimental import pallas as pl
from jax.experimental.pallas import tpu as pltpu
from jax.experimental.pallas import tpu_sc as plsc

PAD = 0

NC = 2
NS = 16
NW = NC * NS

GR = 512


@functools.lru_cache(maxsize=None)
def _make_gather(n_tok: int, vocab: int, d: int):
    per_w = n_tok // NW
    n_group = per_w // GR
    mesh = plsc.VectorSubcoreMesh(core_axis_name="c", subcore_axis_name="s")

    @functools.partial(
        pl.kernel,
        mesh=mesh,
        compiler_params=pltpu.CompilerParams(use_tc_tiling_on_sc=False),
        out_type=jax.ShapeDtypeStruct((n_tok, d), jnp.float32),
        scratch_types=[
            pltpu.VMEM((per_w,), jnp.int32),
            pltpu.VMEM((2, GR, d), jnp.float32),
            pltpu.SemaphoreType.DMA,
            pltpu.SemaphoreType.DMA,
            pltpu.SemaphoreType.DMA,
        ],
    )
    def gather_kernel(tok_hbm, table_hbm, out_hbm, idx_v, rows_v, g0sem, g1sem, osem):
        wid = lax.axis_index("s") * NC + lax.axis_index("c")
        base = wid * per_w
        gsems = (g0sem, g1sem)
        pltpu.sync_copy(tok_hbm.at[wid], idx_v)

        def fire_gathers(g, buf):
            def fire(u, c):
                idx_vec = idx_v[pl.ds(g * GR + u * 16, 16)]
                pltpu.async_copy(
                    table_hbm.at[idx_vec],
                    rows_v.at[buf, pl.ds(u * 16, 16)],
                    gsems[buf],
                )
                return c

            lax.fori_loop(0, GR // 16, fire, 0)

        def wait_gathers(buf):
            pltpu.make_async_copy(
                out_hbm.at[pl.ds(0, GR)], rows_v.at[buf], gsems[buf]
            ).wait()

        def fire_out(g, buf):
            pltpu.async_copy(
                rows_v.at[buf], out_hbm.at[pl.ds(base + g * GR, GR)], osem
            )

        def wait_out():
            pltpu.make_async_copy(
                out_hbm.at[pl.ds(0, GR)], rows_v.at[0], osem
            ).wait()

        fire_gathers(0, 0)
        fire_gathers(1, 1)

        def step(t, carry, last):
            for buf in (0, 1):
                g = 2 * t + buf
                wait_gathers(buf)
                fire_out(g, buf)
                wait_out()
                if not last:
                    fire_gathers(g + 2, buf)
            return carry

        lax.fori_loop(0, n_group // 2 - 1, lambda t, c: step(t, c, False), 0)
        step(n_group // 2 - 1, 0, True)

    return gather_kernel


def kernel(tokens, table):
    b, l = tokens.shape
    vocab, d = table.shape
    n_tok = b * l
    tok_grouped = tokens.reshape(NW, n_tok // NW)
    feat = _make_gather(n_tok, vocab, d)(tok_grouped, table)
    feat = feat.reshape(b, l, d)
    features = jnp.where(jnp.isnan(feat), jnp.float32(0), feat)
    padding_masks = (tokens == PAD)[:, None, None, :]
    sequential_masks = jnp.triu(jnp.ones((l, l), dtype=bool), k=1)
    return features, padding_masks, sequential_masks


# speedup vs baseline: 1.1032x; 1.1032x over previous
import functools

import jax
import jax.numpy as jnp
from jax import lax
from jax.experimental import pallas as pl
from jax.experimental.pallas import tpu as pltpu
from jax.experimental.pallas import tpu_sc as plsc

PAD = 0

NC = 2
NS = 16
NW = NC * NS

GRB = 2


@functools.lru_cache(maxsize=None)
def _make_gather(b: int, l: int, vocab: int, d: int):
    per_w_b = b // NW
    per_w = per_w_b * l
    gr = GRB * l
    n_group = per_w_b // GRB
    mesh = plsc.VectorSubcoreMesh(core_axis_name="c", subcore_axis_name="s")

    @functools.partial(
        pl.kernel,
        mesh=mesh,
        compiler_params=pltpu.CompilerParams(use_tc_tiling_on_sc=False),
        out_type=jax.ShapeDtypeStruct((b, l, d), jnp.float32),
        scratch_types=[
            pltpu.VMEM((per_w,), jnp.int32),
            pltpu.VMEM((2, gr, d), jnp.float32),
            pltpu.SemaphoreType.DMA,
            pltpu.SemaphoreType.DMA,
            pltpu.SemaphoreType.DMA,
        ],
    )
    def gather_kernel(tok_hbm, table_hbm, out_hbm, idx_v, rows_v, g0sem, g1sem, osem):
        wid = lax.axis_index("s") * NC + lax.axis_index("c")
        base_b = wid * per_w_b
        gsems = (g0sem, g1sem)
        pltpu.sync_copy(tok_hbm.at[wid], idx_v)

        def fire_gathers(g, buf):
            def fire(u, c):
                idx_vec = idx_v[pl.ds(g * gr + u * 16, 16)]
                pltpu.async_copy(
                    table_hbm.at[idx_vec],
                    rows_v.at[buf, pl.ds(u * 16, 16)],
                    gsems[buf],
                )
                return c

            lax.fori_loop(0, gr // 16, fire, 0)

        def wait_gathers(buf):
            pltpu.make_async_copy(
                out_hbm.at[pl.ds(0, GRB)], rows_v.at[buf], gsems[buf]
            ).wait()

        def fire_out(g, buf):
            for rb in range(GRB):
                pltpu.async_copy(
                    rows_v.at[buf, pl.ds(rb * l, l)],
                    out_hbm.at[base_b + g * GRB + rb],
                    osem,
                )

        def wait_out():
            for rb in range(GRB):
                pltpu.make_async_copy(
                    out_hbm.at[0], rows_v.at[0, pl.ds(rb * l, l)], osem
                ).wait()

        fire_gathers(0, 0)
        fire_gathers(1, 1)

        def step(t, carry, last):
            for buf in (0, 1):
                g = 2 * t + buf
                wait_gathers(buf)
                fire_out(g, buf)
                wait_out()
                if not last:
                    fire_gathers(g + 2, buf)
            return carry

        lax.fori_loop(0, n_group // 2 - 1, lambda t, c: step(t, c, False), 0)
        step(n_group // 2 - 1, 0, True)

    return gather_kernel


def kernel(tokens, table):
    b, l = tokens.shape
    vocab, d = table.shape
    tok_grouped = tokens.reshape(NW, (b // NW) * l)
    features = _make_gather(b, l, vocab, d)(tok_grouped, table)
    padding_masks = (tokens == PAD)[:, None, None, :]
    sequential_masks = jnp.triu(jnp.ones((l, l), dtype=bool), k=1)
    return features, padding_masks, sequential_masks
